# baseline (speedup 1.0000x reference)
import functools

import jax
import jax.numpy as jnp
from jax import lax
from jax.experimental import pallas as pl
from jax.experimental.pallas import tpu as pltpu
from jax.experimental.pallas import tpu_sc as plsc

_VOCAB = 100000
_EMBED = 64
_BATCH = 1024

_VT = 5120


_RND = 4


@functools.lru_cache(maxsize=None)
def _make_sc_gather(B: int, D: int, V: int):
    info = plsc.get_sparse_core_info()
    nc, ns, L = info.num_cores, info.num_subcores, info.num_lanes
    nw = nc * ns
    assert B % (8 * nw) == 0 and D % L == 0
    bpw = B // nw
    nrounds = bpw // _RND
    mesh = plsc.VectorSubcoreMesh(core_axis_name="c", subcore_axis_name="s")

    @functools.partial(
        pl.kernel,
        mesh=mesh,
        compiler_params=pltpu.CompilerParams(needs_layout_passes=False),
        out_type=jax.ShapeDtypeStruct((B, D), jnp.float32),
        scratch_types=[
            pltpu.VMEM((bpw,), jnp.int32),
            pltpu.VMEM((D, 2 * _RND * 128), jnp.float32),
            pltpu.VMEM((bpw, D), jnp.float32),
            pltpu.SemaphoreType.DMA((2,)),
        ],
    )
    def gather(idx_hbm, et_hbm, out_hbm, idx_v, tbuf, rows_v, sem):
        wid = lax.axis_index("s") * nc + lax.axis_index("c")
        base = wid * bpw
        pltpu.sync_copy(idx_hbm.at[pl.ds(base, bpw)], idx_v)
        dlanes = [lax.iota(jnp.int32, L) + k * L for k in range(D // L)]

        def vvec_of(r, s):
            return idx_v[pl.ds(((r * _RND + s) // L) * L, L)]

        def fire(r):
            half = r % 2
            for s in range(_RND):
                v = vvec_of(r, s)[(r * _RND + s) % L]
                c128 = pl.multiple_of((v >> 7) * 128, 128)
                pltpu.make_async_copy(
                    et_hbm.at[:, pl.ds(c128, 128)],
                    tbuf.at[:, pl.ds((half * _RND + s) * 128, 128)],
                    sem.at[half],
                ).start()

        def drain(r):
            half = r % 2
            for s in range(_RND):
                pltpu.make_async_copy(
                    et_hbm.at[:, pl.ds(0, 128)],
                    tbuf.at[:, pl.ds((half * _RND + s) * 128, 128)],
                    sem.at[half],
                ).wait()

        fire(0)
        for r in range(nrounds):
            if r + 1 < nrounds:
                fire(r + 1)
            drain(r)
            half = r % 2
            for s in range(_RND):
                j = r * _RND + s
                v = vvec_of(r, s)[j % L]
                lane = jnp.full((L,), v & 127, jnp.int32) + (half * _RND + s) * 128
                for k in range(D // L):
                    vals = plsc.load_gather(tbuf, [dlanes[k], lane])
                    plsc.store_scatter(
                        rows_v, [jnp.full((L,), j, jnp.int32), dlanes[k]], vals
                    )
        pltpu.sync_copy(rows_v, out_hbm.at[pl.ds(base, bpw)])

    return gather


def _mm_body(wt_ref, e_ref, b_ref, o_ref):
    acc = lax.dot_general(
        wt_ref[...],
        e_ref[...],
        (((0,), (1,)), ((), ())),
        preferred_element_type=jnp.float32,
    )
    bias = lax.dot_general(
        b_ref[...],
        jnp.ones((e_ref.shape[0], 1), jnp.float32),
        (((0,), (1,)), ((), ())),
        preferred_element_type=jnp.float32,
    )
    o_ref[...] = acc + bias


def _projection_t(wt, e, brow):
    D, V = wt.shape
    B = e.shape[0]
    nt = pl.cdiv(V, _VT)
    return pl.pallas_call(
        _mm_body,
        grid=(nt,),
        in_specs=[
            pl.BlockSpec((D, _VT), lambda i: (0, i)),
            pl.BlockSpec((B, D), lambda i: (0, 0)),
            pl.BlockSpec((1, _VT), lambda i: (0, i)),
        ],
        out_specs=pl.BlockSpec((_VT, B), lambda i: (i, 0)),
        out_shape=jax.ShapeDtypeStruct((V, B), jnp.float32),
        compiler_params=pltpu.CompilerParams(
            dimension_semantics=("arbitrary",),
        ),
    )(wt, e, brow)


def kernel(x, emb_table, W, b):
    idx = x.astype(jnp.int32)
    e = _make_sc_gather(_BATCH, _EMBED, _VOCAB)(idx, emb_table.T)
    out_t = _projection_t(W.T, e, b.reshape(1, _VOCAB))
    return out_t.T


# speedup vs baseline: 1.0029x; 1.0029x over previous
import functools

import jax
import jax.numpy as jnp
from jax import lax
from jax.experimental import pallas as pl
from jax.experimental.pallas import tpu as pltpu
from jax.experimental.pallas import tpu_sc as plsc

_VOCAB = 100000
_EMBED = 64
_BATCH = 1024

_VT = 4096


_RND = 4


@functools.lru_cache(maxsize=None)
def _make_sc_gather(B: int, D: int, V: int):
    info = plsc.get_sparse_core_info()
    nc, ns, L = info.num_cores, info.num_subcores, info.num_lanes
    nw = nc * ns
    assert B % (8 * nw) == 0 and D % L == 0
    bpw = B // nw
    nrounds = bpw // _RND
    mesh = plsc.VectorSubcoreMesh(core_axis_name="c", subcore_axis_name="s")

    @functools.partial(
        pl.kernel,
        mesh=mesh,
        compiler_params=pltpu.CompilerParams(needs_layout_passes=False),
        out_type=jax.ShapeDtypeStruct((B, D), jnp.float32),
        scratch_types=[
            pltpu.VMEM((bpw,), jnp.int32),
            pltpu.VMEM((D, 2 * _RND * 128), jnp.float32),
            pltpu.VMEM((bpw, D), jnp.float32),
            pltpu.SemaphoreType.DMA((2,)),
        ],
    )
    def gather(idx_hbm, et_hbm, out_hbm, idx_v, tbuf, rows_v, sem):
        wid = lax.axis_index("s") * nc + lax.axis_index("c")
        base = wid * bpw
        pltpu.sync_copy(idx_hbm.at[pl.ds(base, bpw)], idx_v)
        dlanes = [lax.iota(jnp.int32, L) + k * L for k in range(D // L)]

        def vvec_of(r, s):
            return idx_v[pl.ds(((r * _RND + s) // L) * L, L)]

        def fire(r):
            half = r % 2
            for s in range(_RND):
                v = vvec_of(r, s)[(r * _RND + s) % L]
                c128 = pl.multiple_of((v >> 7) * 128, 128)
                pltpu.make_async_copy(
                    et_hbm.at[:, pl.ds(c128, 128)],
                    tbuf.at[:, pl.ds((half * _RND + s) * 128, 128)],
                    sem.at[half],
                ).start()

        def drain(r):
            half = r % 2
            for s in range(_RND):
                pltpu.make_async_copy(
                    et_hbm.at[:, pl.ds(0, 128)],
                    tbuf.at[:, pl.ds((half * _RND + s) * 128, 128)],
                    sem.at[half],
                ).wait()

        fire(0)
        for r in range(nrounds):
            if r + 1 < nrounds:
                fire(r + 1)
            drain(r)
            half = r % 2
            for s in range(_RND):
                j = r * _RND + s
                v = vvec_of(r, s)[j % L]
                lane = jnp.full((L,), v & 127, jnp.int32) + (half * _RND + s) * 128
                for k in range(D // L):
                    vals = plsc.load_gather(tbuf, [dlanes[k], lane])
                    plsc.store_scatter(
                        rows_v, [jnp.full((L,), j, jnp.int32), dlanes[k]], vals
                    )
        pltpu.sync_copy(rows_v, out_hbm.at[pl.ds(base, bpw)])

    return gather


def _mm_body(wt_ref, e_ref, b_ref, o_ref):
    acc = lax.dot_general(
        wt_ref[...],
        e_ref[...],
        (((0,), (1,)), ((), ())),
        preferred_element_type=jnp.float32,
    )
    bias = lax.dot_general(
        b_ref[...],
        jnp.ones((e_ref.shape[0], 1), jnp.float32),
        (((0,), (1,)), ((), ())),
        preferred_element_type=jnp.float32,
    )
    o_ref[...] = acc + bias


def _projection_t(wt, e, brow):
    D, V = wt.shape
    B = e.shape[0]
    nt = pl.cdiv(V, _VT)
    return pl.pallas_call(
        _mm_body,
        grid=(nt,),
        in_specs=[
            pl.BlockSpec((D, _VT), lambda i: (0, i)),
            pl.BlockSpec((B, D), lambda i: (0, 0)),
            pl.BlockSpec((1, _VT), lambda i: (0, i)),
        ],
        out_specs=pl.BlockSpec((_VT, B), lambda i: (i, 0)),
        out_shape=jax.ShapeDtypeStruct((V, B), jnp.float32),
        compiler_params=pltpu.CompilerParams(
            dimension_semantics=("arbitrary",),
        ),
    )(wt, e, brow)


def kernel(x, emb_table, W, b):
    idx = x.astype(jnp.int32)
    e = _make_sc_gather(_BATCH, _EMBED, _VOCAB)(idx, emb_table.T)
    out_t = _projection_t(W.T, e, b.reshape(1, _VOCAB))
    return out_t.T
